# baseline (speedup 1.0000x reference)
import jax
import jax.numpy as jnp
from jax.experimental import pallas as pl
from jax.experimental.pallas import tpu as pltpu

B, D, H, W = 4, 96, 384, 384
DC = 8
ND = D // DC


def _body(s_ref, t_ref, m_ref, x_ref, sum_ref, cnt_ref,
          dot_acc, ns_acc, nt_acc):
    b = pl.program_id(0)
    j = pl.program_id(1)

    @pl.when((b == 0) & (j == 0))
    def _init():
        sum_ref[...] = jnp.zeros_like(sum_ref)
        cnt_ref[...] = jnp.zeros_like(cnt_ref)

    s = s_ref[0, 0]
    t = t_ref[0, 0]
    pd = jnp.sum(s * t, axis=0)
    pn = jnp.sum(s * s, axis=0)
    pt = jnp.sum(t * t, axis=0)

    @pl.when(j == 0)
    def _first():
        dot_acc[...] = pd
        ns_acc[...] = pn
        nt_acc[...] = pt

    @pl.when(j > 0)
    def _rest():
        dot_acc[...] += pd
        ns_acc[...] += pn
        nt_acc[...] += pt

    @pl.when(j == ND - 1)
    def _finish():
        denom = jnp.maximum(jnp.sqrt(ns_acc[...]) * jnp.sqrt(nt_acc[...]),
                            1e-8)
        loss_map = 1.0 - dot_acc[...] / denom
        valid = (x_ref[0] != 0.0) & (m_ref[0] == 0)
        vf = valid.astype(jnp.float32)
        sum_ref[...] += jnp.sum(loss_map * vf, keepdims=True).reshape(1, 1)
        cnt_ref[...] += jnp.sum(vf, keepdims=True).reshape(1, 1)


def kernel(student_feats, teacher_feats, mask, original_x):
    s = student_feats.reshape(B, ND, DC, H, W)
    t = teacher_feats.reshape(B, ND, DC, H, W)
    m = mask.astype(jnp.int8)
    x = original_x.reshape(B, H, W)

    sums, cnts = pl.pallas_call(
        _body,
        grid=(B, ND),
        in_specs=[
            pl.BlockSpec((1, 1, DC, H, W), lambda b, j: (b, j, 0, 0, 0)),
            pl.BlockSpec((1, 1, DC, H, W), lambda b, j: (b, j, 0, 0, 0)),
            pl.BlockSpec((1, H, W), lambda b, j: (b, 0, 0)),
            pl.BlockSpec((1, H, W), lambda b, j: (b, 0, 0)),
        ],
        out_specs=[
            pl.BlockSpec((1, 1), lambda b, j: (0, 0)),
            pl.BlockSpec((1, 1), lambda b, j: (0, 0)),
        ],
        out_shape=[
            jax.ShapeDtypeStruct((1, 1), jnp.float32),
            jax.ShapeDtypeStruct((1, 1), jnp.float32),
        ],
        scratch_shapes=[
            pltpu.VMEM((H, W), jnp.float32),
            pltpu.VMEM((H, W), jnp.float32),
            pltpu.VMEM((H, W), jnp.float32),
        ],
        compiler_params=pltpu.CompilerParams(
            dimension_semantics=("arbitrary", "arbitrary"),
        ),
    )(s, t, m, x)

    return sums[0, 0] / cnts[0, 0]


# speedup vs baseline: 1.0356x; 1.0356x over previous
import jax
import jax.numpy as jnp
from jax.experimental import pallas as pl
from jax.experimental.pallas import tpu as pltpu

B, D, H, W = 4, 96, 384, 384
DC = 12
ND = D // DC


def _body(s_ref, t_ref, m_ref, x_ref, sum_ref, cnt_ref,
          dot_acc, ns_acc, nt_acc):
    b = pl.program_id(0)
    j = pl.program_id(1)

    @pl.when((b == 0) & (j == 0))
    def _init():
        sum_ref[...] = jnp.zeros_like(sum_ref)
        cnt_ref[...] = jnp.zeros_like(cnt_ref)

    s = s_ref[0, 0]
    t = t_ref[0, 0]
    pd = jnp.sum(s * t, axis=0)
    pn = jnp.sum(s * s, axis=0)
    pt = jnp.sum(t * t, axis=0)

    @pl.when(j == 0)
    def _first():
        dot_acc[...] = pd
        ns_acc[...] = pn
        nt_acc[...] = pt

    @pl.when(j > 0)
    def _rest():
        dot_acc[...] += pd
        ns_acc[...] += pn
        nt_acc[...] += pt

    @pl.when(j == ND - 1)
    def _finish():
        denom = jnp.maximum(jnp.sqrt(ns_acc[...]) * jnp.sqrt(nt_acc[...]),
                            1e-8)
        loss_map = 1.0 - dot_acc[...] / denom
        valid = (x_ref[0] != 0.0) & (m_ref[0] == 0)
        vf = valid.astype(jnp.float32)
        sum_ref[...] += jnp.sum(loss_map * vf, keepdims=True).reshape(1, 1)
        cnt_ref[...] += jnp.sum(vf, keepdims=True).reshape(1, 1)


def kernel(student_feats, teacher_feats, mask, original_x):
    s = student_feats.reshape(B, ND, DC, H, W)
    t = teacher_feats.reshape(B, ND, DC, H, W)
    m = mask.astype(jnp.int8)
    x = original_x.reshape(B, H, W)

    sums, cnts = pl.pallas_call(
        _body,
        grid=(B, ND),
        in_specs=[
            pl.BlockSpec((1, 1, DC, H, W), lambda b, j: (b, j, 0, 0, 0)),
            pl.BlockSpec((1, 1, DC, H, W), lambda b, j: (b, j, 0, 0, 0)),
            pl.BlockSpec((1, H, W), lambda b, j: (b, 0, 0)),
            pl.BlockSpec((1, H, W), lambda b, j: (b, 0, 0)),
        ],
        out_specs=[
            pl.BlockSpec((1, 1), lambda b, j: (0, 0)),
            pl.BlockSpec((1, 1), lambda b, j: (0, 0)),
        ],
        out_shape=[
            jax.ShapeDtypeStruct((1, 1), jnp.float32),
            jax.ShapeDtypeStruct((1, 1), jnp.float32),
        ],
        scratch_shapes=[
            pltpu.VMEM((H, W), jnp.float32),
            pltpu.VMEM((H, W), jnp.float32),
            pltpu.VMEM((H, W), jnp.float32),
        ],
        compiler_params=pltpu.CompilerParams(
            dimension_semantics=("arbitrary", "arbitrary"),
        ),
    )(s, t, m, x)

    return sums[0, 0] / cnts[0, 0]
